# baseline (speedup 1.0000x reference)
import functools

import jax
import jax.numpy as jnp
from jax import lax
from jax.experimental import pallas as pl
from jax.experimental.pallas import tpu as pltpu
from jax.experimental.pallas import tpu_sc as plsc

_B, _N, _C = 128, 1024, 96
_NCLS = 1000
_K = 256
_NC, _NS, _L = 2, 16, 16
_NW = _NC * _NS
_RPW = _B // _NW
_CCH = _C // _L


def _select_weights(r):
    kk = jnp.int32(_K)
    ib = lax.bitcast_convert_type(r, jnp.int32)
    key = jnp.where(ib < 0, ib ^ jnp.int32(0x7FFFFFFF), ib)

    cnt0 = jnp.sum((key >= 0).astype(jnp.int32), axis=1, keepdims=True)
    prefix = jnp.where(cnt0 >= kk, jnp.int32(0), jnp.int32(-2147483648))

    def step(j, p):
        bit = jnp.int32(1) << (jnp.int32(30) - j)
        cand = p | bit
        cnt = jnp.sum((key >= cand).astype(jnp.int32), axis=1, keepdims=True)
        return jnp.where(cnt >= kk, cand, p)

    t = lax.fori_loop(0, 31, step, prefix)

    gt = key > t
    tie = key == t
    n_gt = jnp.sum(gt.astype(jnp.int32), axis=1, keepdims=True)
    need = kk - n_gt

    idx = lax.broadcasted_iota(jnp.int32, r.shape, 1)
    key2 = jnp.where(tie, jnp.int32(_N - 1) - idx, jnp.int32(-1))
    p2 = jnp.zeros_like(need)

    def step2(j, p):
        bit = jnp.int32(1) << (jnp.int32(9) - j)
        cand = p | bit
        cnt = jnp.sum((key2 >= cand).astype(jnp.int32), axis=1, keepdims=True)
        return jnp.where(cnt >= need, cand, p)

    p2 = lax.fori_loop(0, 10, step2, p2)
    sel = gt | (key2 >= p2)

    m = jnp.max(r, axis=1, keepdims=True)
    e = jnp.where(sel, jnp.exp(r - m), jnp.float32(0))
    z = jnp.sum(e, axis=1, keepdims=True)
    return e / z


def _weights_body(rel_ref, w_ref):
    w_ref[...] = _select_weights(rel_ref[...])


def _fc_body(feat_ref, fcw_ref, fcb_ref, out_ref):
    logits = lax.dot_general(
        feat_ref[...], fcw_ref[...], (((1,), (1,)), ((), ())),
        preferred_element_type=jnp.float32)
    out_ref[...] = logits + fcb_ref[...]


_mesh = plsc.VectorSubcoreMesh(core_axis_name="c", subcore_axis_name="s")


@functools.partial(
    pl.kernel,
    out_type=jax.ShapeDtypeStruct((_B, _C), jnp.float32),
    mesh=_mesh,
    compiler_params=pltpu.CompilerParams(use_tc_tiling_on_sc=False,
                                         needs_layout_passes=False),
    scratch_types=[
        pltpu.VMEM((_N,), jnp.float32),
        pltpu.VMEM((_K + _L,), jnp.int32),
        pltpu.VMEM((_K + _L,), jnp.float32),
        pltpu.VMEM((_K, _C), jnp.float32),
        pltpu.VMEM((_C,), jnp.float32),
        pltpu.SemaphoreType.DMA,
    ],
)
def _sc_gather_feat(w_hbm, tok_hbm, out_hbm, wrow, cidx, cw, rows, fbuf,
                    sem):
    wid = lax.axis_index("s") * _NC + lax.axis_index("c")
    zeros16 = jnp.zeros((_L,), jnp.int32)
    for j in range((_K + _L) // _L):
        cidx[pl.ds(j * _L, _L)] = zeros16

    for i in range(_RPW):
        b = wid * _RPW + i
        pltpu.sync_copy(w_hbm.at[b], wrow)

        ones = jnp.full((_L,), 1, jnp.int32)
        trash = jnp.full((_L,), _K + 8, jnp.int32)

        def chunk(j, cursor):
            wv = wrow[pl.ds(j * _L, _L)]
            m32 = jnp.minimum(plsc.bitcast(wv, jnp.int32), ones)
            idxv = lax.iota(jnp.int32, _L) + jnp.full(
                (_L,), j * _L + b * _N, jnp.int32)
            pos = plsc.cumsum(m32) - m32 + jnp.full((_L,), cursor, jnp.int32)
            pos = m32 * (pos - trash) + trash
            plsc.store_scatter(cidx, [pos], idxv)
            plsc.store_scatter(cw, [pos], wv)
            return cursor + jnp.sum(m32)

        lax.fori_loop(0, _N // _L, chunk, jnp.int32(0))

        g0 = pltpu.async_copy(tok_hbm.at[cidx.at[pl.ds(0, 128)]],
                              rows.at[pl.ds(0, 128)], sem)
        g1 = pltpu.async_copy(tok_hbm.at[cidx.at[pl.ds(128, 128)]],
                              rows.at[pl.ds(128, 128)], sem)
        g0.wait()
        g1.wait()

        def acc_body(jc, accs):
            wv = cw[pl.ds(jc * _L, _L)]
            for l in range(_L):
                wj = jnp.full((_L,), wv[l], jnp.float32)
                j = jc * _L + l
                accs = tuple(accs[c] + wj * rows[j, pl.ds(c * _L, _L)]
                             for c in range(_CCH))
            return accs

        accs = lax.fori_loop(
            0, _K // _L, acc_body,
            tuple(jnp.zeros((_L,), jnp.float32) for _ in range(_CCH)))
        for c in range(_CCH):
            fbuf[pl.ds(c * _L, _L)] = accs[c]
        pltpu.sync_copy(fbuf, out_hbm.at[b])


def kernel(tokens, reliability, fc_w, fc_b):
    w = pl.pallas_call(
        _weights_body,
        out_shape=jax.ShapeDtypeStruct((_B, _N), jnp.float32),
    )(reliability)
    feat = _sc_gather_feat(w, tokens.reshape(_B * _N, _C))
    logits = pl.pallas_call(
        _fc_body,
        out_shape=jax.ShapeDtypeStruct((_B, _NCLS), jnp.float32),
    )(feat, fc_w, fc_b.reshape(1, _NCLS))
    return logits


# speedup vs baseline: 2.0202x; 2.0202x over previous
import jax
import jax.numpy as jnp
from jax import lax
from jax.experimental import pallas as pl

_B, _N, _C = 128, 1024, 96
_NCLS = 1000
_K = 256
_BB = 8


def _select_weights(r):
    kk = jnp.int32(_K)
    ib = lax.bitcast_convert_type(r, jnp.int32)
    key = jnp.where(ib < 0, ib ^ jnp.int32(0x7FFFFFFF), ib)

    cnt0 = jnp.sum((key >= 0).astype(jnp.int32), axis=1, keepdims=True)
    prefix = jnp.where(cnt0 >= kk, jnp.int32(0), jnp.int32(-2147483648))

    def step(j, p):
        bit = jnp.int32(1) << (jnp.int32(30) - j)
        cand = p | bit
        cnt = jnp.sum((key >= cand).astype(jnp.int32), axis=1, keepdims=True)
        return jnp.where(cnt >= kk, cand, p)

    t = lax.fori_loop(0, 31, step, prefix)

    gt = key > t
    tie = key == t
    n_gt = jnp.sum(gt.astype(jnp.int32), axis=1, keepdims=True)
    need = kk - n_gt

    idx = lax.broadcasted_iota(jnp.int32, r.shape, 1)
    key2 = jnp.where(tie, jnp.int32(_N - 1) - idx, jnp.int32(-1))
    p2 = jnp.zeros_like(need)

    def step2(j, p):
        bit = jnp.int32(1) << (jnp.int32(9) - j)
        cand = p | bit
        cnt = jnp.sum((key2 >= cand).astype(jnp.int32), axis=1, keepdims=True)
        return jnp.where(cnt >= need, cand, p)

    p2 = lax.fori_loop(0, 10, step2, p2)
    sel = gt | (key2 >= p2)

    m = jnp.max(r, axis=1, keepdims=True)
    e = jnp.where(sel, jnp.exp(r - m), jnp.float32(0))
    z = jnp.sum(e, axis=1, keepdims=True)
    return e / z


def _body(rel_ref, tokt_ref, fcw_ref, fcb_ref, out_ref):
    w = _select_weights(rel_ref[...])
    x = tokt_ref[...]
    feat = jnp.sum(x * w[:, None, :], axis=2)
    logits = lax.dot_general(
        feat, fcw_ref[...], (((1,), (1,)), ((), ())),
        preferred_element_type=jnp.float32)
    out_ref[...] = logits + fcb_ref[...]


def kernel(tokens, reliability, fc_w, fc_b):
    tokens_t = jnp.transpose(tokens, (0, 2, 1))
    fcb2 = fc_b.reshape(1, _NCLS)
    return pl.pallas_call(
        _body,
        grid=(_B // _BB,),
        in_specs=[
            pl.BlockSpec((_BB, _N), lambda i: (i, 0)),
            pl.BlockSpec((_BB, _C, _N), lambda i: (i, 0, 0)),
            pl.BlockSpec((_NCLS, _C), lambda i: (0, 0)),
            pl.BlockSpec((1, _NCLS), lambda i: (0, 0)),
        ],
        out_specs=pl.BlockSpec((_BB, _NCLS), lambda i: (i, 0)),
        out_shape=jax.ShapeDtypeStruct((_B, _NCLS), jnp.float32),
    )(reliability, tokens_t, fc_w, fcb2)


# speedup vs baseline: 4.8670x; 2.4091x over previous
import jax
import jax.numpy as jnp
from jax import lax
from jax.experimental import pallas as pl

_B, _N, _C = 128, 1024, 96
_NCLS = 1000
_K = 256
_BB = 32


def _select_weights(r):
    kk = jnp.int32(_K)
    ib = lax.bitcast_convert_type(r, jnp.int32)
    key = jnp.where(ib < 0, ib ^ jnp.int32(0x7FFFFFFF), ib)

    cnt0 = jnp.sum((key >= 0).astype(jnp.int32), axis=1, keepdims=True)
    prefix = jnp.where(cnt0 >= kk, jnp.int32(0), jnp.int32(-2147483648))

    def step(j, p):
        bit = jnp.int32(1) << (jnp.int32(30) - j)
        cand = p | bit
        cnt = jnp.sum((key >= cand).astype(jnp.int32), axis=1, keepdims=True)
        return jnp.where(cnt >= kk, cand, p)

    t = lax.fori_loop(0, 31, step, prefix)

    gt = key > t
    tie = key == t
    n_gt = jnp.sum(gt.astype(jnp.int32), axis=1, keepdims=True)
    need = kk - n_gt

    idx = lax.broadcasted_iota(jnp.int32, r.shape, 1)
    key2 = jnp.where(tie, jnp.int32(_N - 1) - idx, jnp.int32(-1))
    p2 = jnp.zeros_like(need)

    def step2(j, p):
        bit = jnp.int32(1) << (jnp.int32(9) - j)
        cand = p | bit
        cnt = jnp.sum((key2 >= cand).astype(jnp.int32), axis=1, keepdims=True)
        return jnp.where(cnt >= need, cand, p)

    p2 = lax.fori_loop(0, 10, step2, p2)
    sel = gt | (key2 >= p2)

    m = jnp.max(r, axis=1, keepdims=True)
    e = jnp.where(sel, jnp.exp(r - m), jnp.float32(0))
    z = jnp.sum(e, axis=1, keepdims=True)
    return e / z


def _body(rel_ref, tokt_ref, fcw_ref, fcb_ref, out_ref):
    w = _select_weights(rel_ref[...])
    x = tokt_ref[...]
    feat = jnp.sum(x * w[:, None, :], axis=2)
    logits = lax.dot_general(
        feat, fcw_ref[...], (((1,), (1,)), ((), ())),
        preferred_element_type=jnp.float32)
    out_ref[...] = logits + fcb_ref[...]


def kernel(tokens, reliability, fc_w, fc_b):
    tokens_t = jnp.transpose(tokens, (0, 2, 1))
    fcb2 = fc_b.reshape(1, _NCLS)
    return pl.pallas_call(
        _body,
        grid=(_B // _BB,),
        in_specs=[
            pl.BlockSpec((_BB, _N), lambda i: (i, 0)),
            pl.BlockSpec((_BB, _C, _N), lambda i: (i, 0, 0)),
            pl.BlockSpec((_NCLS, _C), lambda i: (0, 0)),
            pl.BlockSpec((1, _NCLS), lambda i: (0, 0)),
        ],
        out_specs=pl.BlockSpec((_BB, _NCLS), lambda i: (i, 0)),
        out_shape=jax.ShapeDtypeStruct((_B, _NCLS), jnp.float32),
    )(reliability, tokens_t, fc_w, fcb2)


# speedup vs baseline: 5.6166x; 1.1540x over previous
import jax
import jax.numpy as jnp
from jax import lax
from jax.experimental import pallas as pl

_B, _N, _C = 128, 1024, 96
_NCLS = 1000
_K = 256
_BB = 64


def _select_weights(r):
    kk = jnp.int32(_K)
    ib = lax.bitcast_convert_type(r, jnp.int32)
    key = jnp.where(ib < 0, ib ^ jnp.int32(0x7FFFFFFF), ib)

    cnt0 = jnp.sum((key >= 0).astype(jnp.int32), axis=1, keepdims=True)
    prefix = jnp.where(cnt0 >= kk, jnp.int32(0), jnp.int32(-2147483648))

    def step(j, p):
        bit = jnp.int32(1) << (jnp.int32(30) - j)
        cand = p | bit
        cnt = jnp.sum((key >= cand).astype(jnp.int32), axis=1, keepdims=True)
        return jnp.where(cnt >= kk, cand, p)

    t = lax.fori_loop(0, 31, step, prefix)

    gt = key > t
    tie = key == t
    n_gt = jnp.sum(gt.astype(jnp.int32), axis=1, keepdims=True)
    need = kk - n_gt

    idx = lax.broadcasted_iota(jnp.int32, r.shape, 1)
    key2 = jnp.where(tie, jnp.int32(_N - 1) - idx, jnp.int32(-1))
    p2 = jnp.zeros_like(need)

    def step2(j, p):
        bit = jnp.int32(1) << (jnp.int32(9) - j)
        cand = p | bit
        cnt = jnp.sum((key2 >= cand).astype(jnp.int32), axis=1, keepdims=True)
        return jnp.where(cnt >= need, cand, p)

    p2 = lax.fori_loop(0, 10, step2, p2)
    sel = gt | (key2 >= p2)

    m = jnp.max(r, axis=1, keepdims=True)
    e = jnp.where(sel, jnp.exp(r - m), jnp.float32(0))
    z = jnp.sum(e, axis=1, keepdims=True)
    return e / z


def _body(rel_ref, tokt_ref, fcw_ref, fcb_ref, out_ref):
    w = _select_weights(rel_ref[...])
    x = tokt_ref[...]
    feat = jnp.sum(x * w[:, None, :], axis=2)
    logits = lax.dot_general(
        feat, fcw_ref[...], (((1,), (1,)), ((), ())),
        preferred_element_type=jnp.float32)
    out_ref[...] = logits + fcb_ref[...]


def kernel(tokens, reliability, fc_w, fc_b):
    tokens_t = jnp.transpose(tokens, (0, 2, 1))
    fcb2 = fc_b.reshape(1, _NCLS)
    return pl.pallas_call(
        _body,
        grid=(_B // _BB,),
        in_specs=[
            pl.BlockSpec((_BB, _N), lambda i: (i, 0)),
            pl.BlockSpec((_BB, _C, _N), lambda i: (i, 0, 0)),
            pl.BlockSpec((_NCLS, _C), lambda i: (0, 0)),
            pl.BlockSpec((1, _NCLS), lambda i: (0, 0)),
        ],
        out_specs=pl.BlockSpec((_BB, _NCLS), lambda i: (i, 0)),
        out_shape=jax.ShapeDtypeStruct((_B, _NCLS), jnp.float32),
    )(reliability, tokens_t, fc_w, fcb2)
